# baseline (speedup 1.0000x reference)
import functools

import jax
import jax.numpy as jnp
from jax import lax
from jax.experimental import pallas as pl
from jax.experimental.pallas import tpu as pltpu
from jax.experimental.pallas import tpu_sc as plsc

IN_DIM = 1000
EMBED = 512
HID = 512
T = 200
B = 128
N = T * B
VPAD = 1008

def _table_body(emb_ref, w_ref, bias_ref, out_ref):
    out_ref[...] = (
        jnp.dot(emb_ref[...], w_ref[...], preferred_element_type=jnp.float32)
        + bias_ref[...]
    )


def _make_table(emb_pad, w_ih_t, bias2d):
    return pl.pallas_call(
        _table_body,
        out_shape=jax.ShapeDtypeStruct((VPAD, HID), jnp.float32),
    )(emb_pad, w_ih_t, bias2d)


_NC = 2
_NS = 16
_NW = _NC * _NS
_BPW = N // _NW
_CH = 80
_NCHUNK = _BPW // _CH


@functools.cache
def _build_gather():
    mesh = plsc.VectorSubcoreMesh(core_axis_name="c", subcore_axis_name="s")

    @functools.partial(
        pl.kernel,
        out_type=jax.ShapeDtypeStruct((N, HID), jnp.float32),
        mesh=mesh,
        scratch_types=[
            pltpu.VMEM((_BPW,), jnp.int32),
            pltpu.VMEM((_CH, HID), jnp.float32),
            pltpu.VMEM((_CH, HID), jnp.float32),
            pltpu.SemaphoreType.DMA,
            pltpu.SemaphoreType.DMA,
        ],
    )
    def _gather(table_hbm, idx_hbm, out_hbm, idx_v, buf0, buf1, sem0, sem1):
        wid = lax.axis_index("s") * _NC + lax.axis_index("c")
        base = wid * _BPW
        pltpu.sync_copy(idx_hbm.at[pl.ds(base, _BPW)], idx_v)
        bufs = (buf0, buf1)
        sems = (sem0, sem1)
        copies = [
            pltpu.async_copy(
                table_hbm.at[idx_v.at[pl.ds(0, _CH)]], bufs[0], sems[0]
            )
        ]
        for c in range(_NCHUNK):
            if c + 1 < _NCHUNK:
                copies.append(
                    pltpu.async_copy(
                        table_hbm.at[idx_v.at[pl.ds((c + 1) * _CH, _CH)]],
                        bufs[(c + 1) % 2],
                        sems[(c + 1) % 2],
                    )
                )
            copies[c].wait()
            pltpu.sync_copy(bufs[c % 2], out_hbm.at[pl.ds(base + c * _CH, _CH)])

    return _gather


def _rnn_body(len_ref, xp_ref, whh_ref, out_ref, h_ref):
    t = pl.program_id(0)

    @pl.when(t == 0)
    def _():
        h_ref[...] = jnp.zeros_like(h_ref)

    h = h_ref[...]
    h_new = jnp.maximum(
        xp_ref[...]
        + jnp.dot(
            h.astype(jnp.bfloat16), whh_ref[...],
            preferred_element_type=jnp.float32,
        ),
        0.0,
    )
    mask = len_ref[...] > t
    h_ref[...] = jnp.where(mask, h_new, h)
    out_ref[...] = jnp.where(mask, h_new, 0.0).astype(jnp.bfloat16)


def _run_rnn(lengths2d, xp, w_hh_t_bf16):
    return pl.pallas_call(
        _rnn_body,
        grid=(T,),
        in_specs=[
            pl.BlockSpec((B, 1), lambda t: (0, 0)),
            pl.BlockSpec((B, HID), lambda t: (t, 0)),
            pl.BlockSpec((HID, HID), lambda t: (0, 0)),
        ],
        out_specs=pl.BlockSpec((B, HID), lambda t: (t, 0)),
        out_shape=jax.ShapeDtypeStruct((N, HID), jnp.bfloat16),
        scratch_shapes=[pltpu.VMEM((B, HID), jnp.float32)],
    )(lengths2d, xp, w_hh_t_bf16)


_TT = 2
_RT = _TT * B

def _proj_body(rec_ref, w_ref, b_ref, out_ref):
    logits = (
        jnp.dot(rec_ref[...], w_ref[...], preferred_element_type=jnp.float32)
        + b_ref[...]
    )
    m = jnp.max(logits, axis=-1, keepdims=True)
    e = jnp.exp(logits - m)
    s = jnp.sum(e, axis=-1, keepdims=True)
    out_ref[...] = (logits - m - jnp.log(s)).reshape(_TT, B, IN_DIM)


def _run_proj(rec_flat, w_out_t_bf16, b_out2d):
    return pl.pallas_call(
        _proj_body,
        grid=(T // _TT,),
        in_specs=[
            pl.BlockSpec((_RT, HID), lambda i: (i, 0)),
            pl.BlockSpec((HID, IN_DIM), lambda i: (0, 0)),
            pl.BlockSpec((1, IN_DIM), lambda i: (0, 0)),
        ],
        out_specs=pl.BlockSpec((_TT, B, IN_DIM), lambda i: (i, 0, 0)),
        out_shape=jax.ShapeDtypeStruct((T, B, IN_DIM), jnp.float32),
    )(rec_flat, w_out_t_bf16, b_out2d)


def kernel(inputs, lengths, embed, W_ih, W_hh, b_ih, b_hh, W_out, b_out):
    emb_pad = jnp.pad(embed, ((0, VPAD - (IN_DIM + 1)), (0, 0)))
    bias2d = (b_ih + b_hh).reshape(1, HID)
    table = _make_table(emb_pad, W_ih.T, bias2d)

    idx = inputs.reshape(N).astype(jnp.int32)
    xp = _build_gather()(table, idx)

    lengths2d = lengths.reshape(B, 1).astype(jnp.int32)
    rec = _run_rnn(lengths2d, xp, W_hh.T.astype(jnp.bfloat16))

    return _run_proj(rec, W_out.T.astype(jnp.bfloat16), b_out.reshape(1, IN_DIM))


# speedup vs baseline: 1.5177x; 1.5177x over previous
import functools

import jax
import jax.numpy as jnp
from jax import lax
from jax.experimental import pallas as pl
from jax.experimental.pallas import tpu as pltpu
from jax.experimental.pallas import tpu_sc as plsc

IN_DIM = 1000
EMBED = 512
HID = 512
T = 200
B = 128
N = T * B
VPAD = 1008

def _table_body(emb_ref, w_ref, bias_ref, out_ref):
    out_ref[...] = (
        jnp.dot(emb_ref[...], w_ref[...], preferred_element_type=jnp.float32)
        + bias_ref[...]
    )


def _make_table(emb_pad, w_ih_t, bias2d):
    return pl.pallas_call(
        _table_body,
        out_shape=jax.ShapeDtypeStruct((VPAD, HID), jnp.float32),
    )(emb_pad, w_ih_t, bias2d)


_NC = 2
_NS = 16
_NW = _NC * _NS
_BPW = N // _NW
_CH = 80
_NCHUNK = _BPW // _CH


@functools.cache
def _build_gather():
    mesh = plsc.VectorSubcoreMesh(core_axis_name="c", subcore_axis_name="s")

    @functools.partial(
        pl.kernel,
        out_type=jax.ShapeDtypeStruct((N, HID), jnp.float32),
        mesh=mesh,
        scratch_types=[
            pltpu.VMEM((_BPW,), jnp.int32),
            pltpu.VMEM((_CH, HID), jnp.float32),
            pltpu.VMEM((_CH, HID), jnp.float32),
            pltpu.SemaphoreType.DMA,
            pltpu.SemaphoreType.DMA,
        ],
    )
    def _gather(table_hbm, idx_hbm, out_hbm, idx_v, buf0, buf1, sem0, sem1):
        wid = lax.axis_index("s") * _NC + lax.axis_index("c")
        base = wid * _BPW
        pltpu.sync_copy(idx_hbm.at[pl.ds(base, _BPW)], idx_v)
        bufs = (buf0, buf1)
        sems = (sem0, sem1)
        copies = [
            pltpu.async_copy(
                table_hbm.at[idx_v.at[pl.ds(0, _CH)]], bufs[0], sems[0]
            )
        ]
        for c in range(_NCHUNK):
            if c + 1 < _NCHUNK:
                copies.append(
                    pltpu.async_copy(
                        table_hbm.at[idx_v.at[pl.ds((c + 1) * _CH, _CH)]],
                        bufs[(c + 1) % 2],
                        sems[(c + 1) % 2],
                    )
                )
            copies[c].wait()
            pltpu.sync_copy(bufs[c % 2], out_hbm.at[pl.ds(base + c * _CH, _CH)])

    return _gather


_UT = 2

def _rnn_body(len_ref, xp_ref, whh_ref, out_ref, h_ref):
    i = pl.program_id(0)

    @pl.when(i == 0)
    def _():
        h_ref[...] = jnp.zeros_like(h_ref)

    h = h_ref[...]
    for j in range(_UT):
        t = i * _UT + j
        h_new = jnp.maximum(
            xp_ref[pl.ds(j * B, B), :]
            + jnp.dot(
                h.astype(jnp.bfloat16), whh_ref[...],
                preferred_element_type=jnp.float32,
            ),
            0.0,
        )
        mask = len_ref[...] > t
        h = jnp.where(mask, h_new, h)
        out_ref[pl.ds(j * B, B), :] = jnp.where(mask, h_new, 0.0).astype(
            jnp.bfloat16
        )
    h_ref[...] = h


def _run_rnn(lengths2d, xp, w_hh_t_bf16):
    return pl.pallas_call(
        _rnn_body,
        grid=(T // _UT,),
        in_specs=[
            pl.BlockSpec((B, 1), lambda i: (0, 0)),
            pl.BlockSpec((_UT * B, HID), lambda i: (i, 0)),
            pl.BlockSpec((HID, HID), lambda i: (0, 0)),
        ],
        out_specs=pl.BlockSpec((_UT * B, HID), lambda i: (i, 0)),
        out_shape=jax.ShapeDtypeStruct((N, HID), jnp.bfloat16),
        scratch_shapes=[pltpu.VMEM((B, HID), jnp.float32)],
    )(lengths2d, xp, w_hh_t_bf16)


_TT = 2
_RT = _TT * B

def _proj_body(rec_ref, w_ref, b_ref, out_ref):
    xt = rec_ref[...].T
    logits = (
        jnp.dot(w_ref[...], xt, preferred_element_type=jnp.float32)
        + b_ref[...]
    )
    m = jnp.max(logits, axis=0, keepdims=True)
    e = jnp.exp(logits - m)
    s = jnp.sum(e, axis=0, keepdims=True)
    res = logits - m - jnp.log(s)
    out_ref[0] = res[:, :B]
    out_ref[1] = res[:, B:]


def _run_proj(rec_flat, w_out_bf16, b_out2d):
    return pl.pallas_call(
        _proj_body,
        grid=(T // _TT,),
        in_specs=[
            pl.BlockSpec((_RT, HID), lambda i: (i, 0)),
            pl.BlockSpec((IN_DIM, HID), lambda i: (0, 0)),
            pl.BlockSpec((IN_DIM, 1), lambda i: (0, 0)),
        ],
        out_specs=pl.BlockSpec((_TT, IN_DIM, B), lambda i: (i, 0, 0)),
        out_shape=jax.ShapeDtypeStruct((T, IN_DIM, B), jnp.float32),
    )(rec_flat, w_out_bf16, b_out2d)


def kernel(inputs, lengths, embed, W_ih, W_hh, b_ih, b_hh, W_out, b_out):
    emb_pad = jnp.pad(embed, ((0, VPAD - (IN_DIM + 1)), (0, 0)))
    bias2d = (b_ih + b_hh).reshape(1, HID)
    table = _make_table(emb_pad, W_ih.T, bias2d)

    idx = inputs.reshape(N).astype(jnp.int32)
    xp = _build_gather()(table, idx)

    lengths2d = lengths.reshape(B, 1).astype(jnp.int32)
    rec = _run_rnn(lengths2d, xp, W_hh.T.astype(jnp.bfloat16))

    out_tvb = _run_proj(rec, W_out.astype(jnp.bfloat16), b_out.reshape(IN_DIM, 1))
    return out_tvb.transpose(0, 2, 1)


# speedup vs baseline: 1.9077x; 1.2569x over previous
import functools

import jax
import jax.numpy as jnp
from jax import lax
from jax.experimental import pallas as pl
from jax.experimental.pallas import tpu as pltpu
from jax.experimental.pallas import tpu_sc as plsc

IN_DIM = 1000
EMBED = 512
HID = 512
T = 200
B = 128
N = T * B
VPAD = 1008

def _table_body(emb_ref, w_ref, bias_ref, out_ref):
    out_ref[...] = (
        jnp.dot(emb_ref[...], w_ref[...], preferred_element_type=jnp.float32)
        + bias_ref[...]
    )


def _make_table(emb_pad, w_ih_t, bias2d):
    return pl.pallas_call(
        _table_body,
        out_shape=jax.ShapeDtypeStruct((VPAD, HID), jnp.float32),
    )(emb_pad, w_ih_t, bias2d)


_NC = 2
_NS = 16
_NW = _NC * _NS
_BPW = N // _NW
_CH = 40
_NBUF = 4
_NCHUNK = _BPW // _CH


@functools.cache
def _build_gather():
    mesh = plsc.VectorSubcoreMesh(core_axis_name="c", subcore_axis_name="s")

    @functools.partial(
        pl.kernel,
        out_type=jax.ShapeDtypeStruct((N, HID), jnp.float32),
        mesh=mesh,
        scratch_types=[
            pltpu.VMEM((_BPW,), jnp.int32),
        ]
        + [pltpu.VMEM((_CH, HID), jnp.float32) for _ in range(_NBUF)]
        + [pltpu.SemaphoreType.DMA for _ in range(_NBUF)],
    )
    def _gather(table_hbm, idx_hbm, out_hbm, idx_v, *bufsem):
        bufs = bufsem[:_NBUF]
        sems = bufsem[_NBUF:]
        wid = lax.axis_index("s") * _NC + lax.axis_index("c")
        base = wid * _BPW
        pltpu.sync_copy(idx_hbm.at[pl.ds(base, _BPW)], idx_v)
        copies = [
            pltpu.async_copy(
                table_hbm.at[idx_v.at[pl.ds(c * _CH, _CH)]],
                bufs[c % _NBUF],
                sems[c % _NBUF],
            )
            for c in range(_NBUF)
        ]
        for c in range(_NCHUNK):
            copies[c].wait()
            pltpu.sync_copy(bufs[c % _NBUF], out_hbm.at[pl.ds(base + c * _CH, _CH)])
            nxt = c + _NBUF
            if nxt < _NCHUNK:
                copies.append(
                    pltpu.async_copy(
                        table_hbm.at[idx_v.at[pl.ds(nxt * _CH, _CH)]],
                        bufs[nxt % _NBUF],
                        sems[nxt % _NBUF],
                    )
                )

    return _gather


_UT = 4

def _rnn_body(len_ref, xp_ref, whh_ref, out_ref, h_ref):
    i = pl.program_id(0)

    @pl.when(i == 0)
    def _():
        h_ref[...] = jnp.zeros_like(h_ref)

    h = h_ref[...]
    for j in range(_UT):
        t = i * _UT + j
        h_new = jnp.maximum(
            xp_ref[pl.ds(j * B, B), :]
            + jnp.dot(
                h.astype(jnp.bfloat16), whh_ref[...],
                preferred_element_type=jnp.float32,
            ),
            0.0,
        )
        mask = len_ref[...] > t
        h = jnp.where(mask, h_new, h)
        out_ref[pl.ds(j * B, B), :] = jnp.where(mask, h_new, 0.0).astype(
            jnp.bfloat16
        )
    h_ref[...] = h


def _run_rnn(lengths2d, xp, w_hh_t_bf16):
    return pl.pallas_call(
        _rnn_body,
        grid=(T // _UT,),
        in_specs=[
            pl.BlockSpec((B, 1), lambda i: (0, 0)),
            pl.BlockSpec((_UT * B, HID), lambda i: (i, 0)),
            pl.BlockSpec((HID, HID), lambda i: (0, 0)),
        ],
        out_specs=pl.BlockSpec((_UT * B, HID), lambda i: (i, 0)),
        out_shape=jax.ShapeDtypeStruct((N, HID), jnp.bfloat16),
        scratch_shapes=[pltpu.VMEM((B, HID), jnp.float32)],
    )(lengths2d, xp, w_hh_t_bf16)


_TT = 4
_RT = _TT * B

def _proj_body(rec_ref, w_ref, b_ref, out_ref):
    xt = rec_ref[...].T
    logits = (
        jnp.dot(w_ref[...], xt, preferred_element_type=jnp.float32)
        + b_ref[...]
    )
    m = jnp.max(logits, axis=0, keepdims=True)
    e = jnp.exp(logits - m)
    s = jnp.sum(e, axis=0, keepdims=True)
    res = logits - m - jnp.log(s)
    for j in range(_TT):
        out_ref[j] = res[:, j * B:(j + 1) * B]


def _run_proj(rec_flat, w_out_bf16, b_out2d):
    return pl.pallas_call(
        _proj_body,
        grid=(T // _TT,),
        in_specs=[
            pl.BlockSpec((_RT, HID), lambda i: (i, 0)),
            pl.BlockSpec((IN_DIM, HID), lambda i: (0, 0)),
            pl.BlockSpec((IN_DIM, 1), lambda i: (0, 0)),
        ],
        out_specs=pl.BlockSpec((_TT, IN_DIM, B), lambda i: (i, 0, 0)),
        out_shape=jax.ShapeDtypeStruct((T, IN_DIM, B), jnp.float32),
    )(rec_flat, w_out_bf16, b_out2d)


def kernel(inputs, lengths, embed, W_ih, W_hh, b_ih, b_hh, W_out, b_out):
    emb_pad = jnp.pad(embed, ((0, VPAD - (IN_DIM + 1)), (0, 0)))
    bias2d = (b_ih + b_hh).reshape(1, HID)
    table = _make_table(emb_pad, W_ih.T, bias2d)

    idx = inputs.reshape(N).astype(jnp.int32)
    xp = _build_gather()(table, idx)

    lengths2d = lengths.reshape(B, 1).astype(jnp.int32)
    rec = _run_rnn(lengths2d, xp, W_hh.T.astype(jnp.bfloat16))

    out_tvb = _run_proj(rec, W_out.astype(jnp.bfloat16), b_out.reshape(IN_DIM, 1))
    return out_tvb.transpose(0, 2, 1)


# speedup vs baseline: 2.1052x; 1.1035x over previous
import functools

import jax
import jax.numpy as jnp
from jax import lax
from jax.experimental import pallas as pl
from jax.experimental.pallas import tpu as pltpu
from jax.experimental.pallas import tpu_sc as plsc

IN_DIM = 1000
EMBED = 512
HID = 512
T = 200
B = 128
N = T * B
VPAD = 1008

_HH = HID // 2

def _table_body(emb_ref, w_ref, bias_ref, out_ref):
    res = (
        jnp.dot(emb_ref[...], w_ref[...], preferred_element_type=jnp.float32)
        + bias_ref[...]
    )
    lo = lax.bitcast_convert_type(res[:, :_HH], jnp.uint32)
    hi = lax.bitcast_convert_type(res[:, _HH:], jnp.uint32)
    lo_t = (lo + 0x8000) >> 16
    hi_t = (hi + 0x8000) & jnp.uint32(0xFFFF0000)
    out_ref[...] = lax.bitcast_convert_type(hi_t | lo_t, jnp.float32)


def _make_table(emb_pad, w_ih_t, bias2d):
    return pl.pallas_call(
        _table_body,
        out_shape=jax.ShapeDtypeStruct((VPAD, _HH), jnp.float32),
    )(emb_pad, w_ih_t, bias2d)


_NC = 2
_NS = 16
_NW = _NC * _NS
_BPW = N // _NW
_CH = 80
_NBUF = 4
_NCHUNK = _BPW // _CH


@functools.cache
def _build_gather():
    mesh = plsc.VectorSubcoreMesh(core_axis_name="c", subcore_axis_name="s")

    @functools.partial(
        pl.kernel,
        out_type=jax.ShapeDtypeStruct((N, _HH), jnp.float32),
        mesh=mesh,
        scratch_types=[
            pltpu.VMEM((_BPW,), jnp.int32),
        ]
        + [pltpu.VMEM((_CH, _HH), jnp.float32) for _ in range(_NBUF)]
        + [pltpu.SemaphoreType.DMA for _ in range(_NBUF)],
    )
    def _gather(table_hbm, idx_hbm, out_hbm, idx_v, *bufsem):
        bufs = bufsem[:_NBUF]
        sems = bufsem[_NBUF:]
        wid = lax.axis_index("s") * _NC + lax.axis_index("c")
        base = wid * _BPW
        pltpu.sync_copy(idx_hbm.at[pl.ds(base, _BPW)], idx_v)
        copies = [
            pltpu.async_copy(
                table_hbm.at[idx_v.at[pl.ds(c * _CH, _CH)]],
                bufs[c % _NBUF],
                sems[c % _NBUF],
            )
            for c in range(_NBUF)
        ]
        for c in range(_NCHUNK):
            copies[c].wait()
            pltpu.sync_copy(bufs[c % _NBUF], out_hbm.at[pl.ds(base + c * _CH, _CH)])
            nxt = c + _NBUF
            if nxt < _NCHUNK:
                copies.append(
                    pltpu.async_copy(
                        table_hbm.at[idx_v.at[pl.ds(nxt * _CH, _CH)]],
                        bufs[nxt % _NBUF],
                        sems[nxt % _NBUF],
                    )
                )

    return _gather


_UT = 4

def _rnn_body(len_ref, xp_ref, whh_ref, out_ref, h_ref):
    i = pl.program_id(0)

    @pl.when(i == 0)
    def _():
        h_ref[...] = jnp.zeros_like(h_ref)

    h = h_ref[...]
    for j in range(_UT):
        t = i * _UT + j
        w = lax.bitcast_convert_type(xp_ref[pl.ds(j * B, B), :], jnp.uint32)
        x = jnp.concatenate(
            [
                lax.bitcast_convert_type(w << 16, jnp.float32),
                lax.bitcast_convert_type(w & jnp.uint32(0xFFFF0000), jnp.float32),
            ],
            axis=1,
        )
        h_new = jnp.maximum(
            x
            + jnp.dot(
                h.astype(jnp.bfloat16), whh_ref[...],
                preferred_element_type=jnp.float32,
            ),
            0.0,
        )
        mask = len_ref[...] > t
        h = jnp.where(mask, h_new, h)
        out_ref[pl.ds(j * B, B), :] = jnp.where(mask, h_new, 0.0).astype(
            jnp.bfloat16
        )
    h_ref[...] = h


def _run_rnn(lengths2d, xp, w_hh_t_bf16):
    return pl.pallas_call(
        _rnn_body,
        grid=(T // _UT,),
        in_specs=[
            pl.BlockSpec((B, 1), lambda i: (0, 0)),
            pl.BlockSpec((_UT * B, _HH), lambda i: (i, 0)),
            pl.BlockSpec((HID, HID), lambda i: (0, 0)),
        ],
        out_specs=pl.BlockSpec((_UT * B, HID), lambda i: (i, 0)),
        out_shape=jax.ShapeDtypeStruct((N, HID), jnp.bfloat16),
        scratch_shapes=[pltpu.VMEM((B, HID), jnp.float32)],
    )(lengths2d, xp, w_hh_t_bf16)


_TT = 4
_RT = _TT * B

def _proj_body(rec_ref, w_ref, b_ref, out_ref):
    xt = rec_ref[...].T
    logits = (
        jnp.dot(w_ref[...], xt, preferred_element_type=jnp.float32)
        + b_ref[...]
    )
    m = jnp.max(logits, axis=0, keepdims=True)
    e = jnp.exp(logits - m)
    s = jnp.sum(e, axis=0, keepdims=True)
    res = logits - m - jnp.log(s)
    for j in range(_TT):
        out_ref[j] = res[:, j * B:(j + 1) * B]


def _run_proj(rec_flat, w_out_bf16, b_out2d):
    return pl.pallas_call(
        _proj_body,
        grid=(T // _TT,),
        in_specs=[
            pl.BlockSpec((_RT, HID), lambda i: (i, 0)),
            pl.BlockSpec((IN_DIM, HID), lambda i: (0, 0)),
            pl.BlockSpec((IN_DIM, 1), lambda i: (0, 0)),
        ],
        out_specs=pl.BlockSpec((_TT, IN_DIM, B), lambda i: (i, 0, 0)),
        out_shape=jax.ShapeDtypeStruct((T, IN_DIM, B), jnp.float32),
    )(rec_flat, w_out_bf16, b_out2d)


def kernel(inputs, lengths, embed, W_ih, W_hh, b_ih, b_hh, W_out, b_out):
    emb_pad = jnp.pad(embed, ((0, VPAD - (IN_DIM + 1)), (0, 0)))
    bias2d = (b_ih + b_hh).reshape(1, HID)
    table = _make_table(emb_pad, W_ih.T, bias2d)

    idx = inputs.reshape(N).astype(jnp.int32)
    xp = _build_gather()(table, idx)

    lengths2d = lengths.reshape(B, 1).astype(jnp.int32)
    rec = _run_rnn(lengths2d, xp, W_hh.T.astype(jnp.bfloat16))

    out_tvb = _run_proj(rec, W_out.astype(jnp.bfloat16), b_out.reshape(IN_DIM, 1))
    return out_tvb.transpose(0, 2, 1)


# speedup vs baseline: 2.5243x; 1.1991x over previous
import functools

import jax
import jax.numpy as jnp
from jax import lax
from jax.experimental import pallas as pl
from jax.experimental.pallas import tpu as pltpu
from jax.experimental.pallas import tpu_sc as plsc

IN_DIM = 1000
EMBED = 512
HID = 512
T = 200
B = 128
N = T * B
VPAD = 1008

_HH = HID // 2

def _table_body(emb_ref, w_ref, bias_ref, out_ref):
    res = (
        jnp.dot(emb_ref[...], w_ref[...], preferred_element_type=jnp.float32)
        + bias_ref[...]
    )
    lo = lax.bitcast_convert_type(res[:, :_HH], jnp.uint32)
    hi = lax.bitcast_convert_type(res[:, _HH:], jnp.uint32)
    lo_t = (lo + 0x8000) >> 16
    hi_t = (hi + 0x8000) & jnp.uint32(0xFFFF0000)
    out_ref[...] = lax.bitcast_convert_type(hi_t | lo_t, jnp.float32)


def _make_table(emb_pad, w_ih_t, bias2d):
    return pl.pallas_call(
        _table_body,
        out_shape=jax.ShapeDtypeStruct((VPAD, _HH), jnp.float32),
    )(emb_pad, w_ih_t, bias2d)


_NC = 2
_NS = 16
_NW = _NC * _NS
_BPW = N // _NW
_CH = 80
_NBUF = 4
_NCHUNK = _BPW // _CH


@functools.cache
def _build_gather():
    mesh = plsc.VectorSubcoreMesh(core_axis_name="c", subcore_axis_name="s")

    @functools.partial(
        pl.kernel,
        out_type=jax.ShapeDtypeStruct((N, _HH), jnp.float32),
        mesh=mesh,
        scratch_types=[
            pltpu.VMEM((_BPW,), jnp.int32),
        ]
        + [pltpu.VMEM((_CH, _HH), jnp.float32) for _ in range(_NBUF)]
        + [pltpu.SemaphoreType.DMA for _ in range(_NBUF)],
    )
    def _gather(table_hbm, idx_hbm, out_hbm, idx_v, *bufsem):
        bufs = bufsem[:_NBUF]
        sems = bufsem[_NBUF:]
        wid = lax.axis_index("s") * _NC + lax.axis_index("c")
        base = wid * _BPW
        pltpu.sync_copy(idx_hbm.at[pl.ds(base, _BPW)], idx_v)
        copies = [
            pltpu.async_copy(
                table_hbm.at[idx_v.at[pl.ds(c * _CH, _CH)]],
                bufs[c % _NBUF],
                sems[c % _NBUF],
            )
            for c in range(_NBUF)
        ]
        for c in range(_NCHUNK):
            copies[c].wait()
            pltpu.sync_copy(bufs[c % _NBUF], out_hbm.at[pl.ds(base + c * _CH, _CH)])
            nxt = c + _NBUF
            if nxt < _NCHUNK:
                copies.append(
                    pltpu.async_copy(
                        table_hbm.at[idx_v.at[pl.ds(nxt * _CH, _CH)]],
                        bufs[nxt % _NBUF],
                        sems[nxt % _NBUF],
                    )
                )

    return _gather


_UT = 4

def _fused_body(len_ref, xp_ref, whh_ref, wout_ref, b_ref, out_ref, h_ref):
    i = pl.program_id(0)

    @pl.when(i == 0)
    def _():
        h_ref[...] = jnp.zeros_like(h_ref)

    h = h_ref[...]
    recs = []
    for j in range(_UT):
        t = i * _UT + j
        w = lax.bitcast_convert_type(xp_ref[pl.ds(j * B, B), :], jnp.uint32)
        x = jnp.concatenate(
            [
                lax.bitcast_convert_type(w << 16, jnp.float32),
                lax.bitcast_convert_type(w & jnp.uint32(0xFFFF0000), jnp.float32),
            ],
            axis=1,
        )
        h_new = jnp.maximum(
            x
            + jnp.dot(
                h.astype(jnp.bfloat16), whh_ref[...],
                preferred_element_type=jnp.float32,
            ),
            0.0,
        )
        mask = len_ref[...] > t
        h = jnp.where(mask, h_new, h)
        recs.append(jnp.where(mask, h_new, 0.0).astype(jnp.bfloat16))
    h_ref[...] = h

    rec = jnp.concatenate(recs, axis=0)
    xt = rec.T
    logits = (
        jnp.dot(wout_ref[...], xt, preferred_element_type=jnp.float32)
        + b_ref[...]
    )
    m = jnp.max(logits, axis=0, keepdims=True)
    e = jnp.exp(logits - m)
    s = jnp.sum(e, axis=0, keepdims=True)
    res = logits - m - jnp.log(s)
    for j in range(_UT):
        out_ref[j] = res[:, j * B:(j + 1) * B]


def _run_fused(lengths2d, xp, w_hh_t_bf16, w_out_bf16, b_out2d):
    return pl.pallas_call(
        _fused_body,
        grid=(T // _UT,),
        in_specs=[
            pl.BlockSpec((B, 1), lambda i: (0, 0)),
            pl.BlockSpec((_UT * B, _HH), lambda i: (i, 0)),
            pl.BlockSpec((HID, HID), lambda i: (0, 0)),
            pl.BlockSpec((IN_DIM, HID), lambda i: (0, 0)),
            pl.BlockSpec((IN_DIM, 1), lambda i: (0, 0)),
        ],
        out_specs=pl.BlockSpec((_UT, IN_DIM, B), lambda i: (i, 0, 0)),
        out_shape=jax.ShapeDtypeStruct((T, IN_DIM, B), jnp.float32),
        scratch_shapes=[pltpu.VMEM((B, HID), jnp.float32)],
    )(lengths2d, xp, w_hh_t_bf16, w_out_bf16, b_out2d)


def kernel(inputs, lengths, embed, W_ih, W_hh, b_ih, b_hh, W_out, b_out):
    emb_pad = jnp.pad(embed, ((0, VPAD - (IN_DIM + 1)), (0, 0)))
    bias2d = (b_ih + b_hh).reshape(1, HID)
    table = _make_table(emb_pad, W_ih.T, bias2d)

    idx = inputs.reshape(N).astype(jnp.int32)
    xp = _build_gather()(table, idx)

    lengths2d = lengths.reshape(B, 1).astype(jnp.int32)
    out_tvb = _run_fused(
        lengths2d, xp, W_hh.T.astype(jnp.bfloat16),
        W_out.astype(jnp.bfloat16), b_out.reshape(IN_DIM, 1),
    )
    return out_tvb.transpose(0, 2, 1)


# speedup vs baseline: 2.7072x; 1.0725x over previous
import functools

import jax
import jax.numpy as jnp
from jax import lax
from jax.experimental import pallas as pl
from jax.experimental.pallas import tpu as pltpu
from jax.experimental.pallas import tpu_sc as plsc

IN_DIM = 1000
EMBED = 512
HID = 512
T = 200
B = 128
N = T * B
VPAD = 1008

_HH = HID // 2

def _table_body(emb_ref, w_ref, bias_ref, out_ref):
    res = (
        jnp.dot(emb_ref[...], w_ref[...], preferred_element_type=jnp.float32)
        + bias_ref[...]
    )
    lo = lax.bitcast_convert_type(res[:, :_HH], jnp.uint32)
    hi = lax.bitcast_convert_type(res[:, _HH:], jnp.uint32)
    lo_t = (lo + 0x8000) >> 16
    hi_t = (hi + 0x8000) & jnp.uint32(0xFFFF0000)
    out_ref[...] = lax.bitcast_convert_type(hi_t | lo_t, jnp.float32)


def _make_table(emb_pad, w_ih_t, bias2d):
    return pl.pallas_call(
        _table_body,
        out_shape=jax.ShapeDtypeStruct((VPAD, _HH), jnp.float32),
    )(emb_pad, w_ih_t, bias2d)


_NC = 2
_NS = 16
_NW = _NC * _NS
_BPW = N // _NW
_CH = 80
_NBUF = 4
_NCHUNK = _BPW // _CH


@functools.cache
def _build_gather():
    mesh = plsc.VectorSubcoreMesh(core_axis_name="c", subcore_axis_name="s")

    @functools.partial(
        pl.kernel,
        out_type=jax.ShapeDtypeStruct((N, _HH), jnp.float32),
        mesh=mesh,
        scratch_types=[
            pltpu.VMEM((_BPW,), jnp.int32),
        ]
        + [pltpu.VMEM((_CH, _HH), jnp.float32) for _ in range(_NBUF)]
        + [pltpu.SemaphoreType.DMA for _ in range(_NBUF)],
    )
    def _gather(table_hbm, idx_hbm, out_hbm, idx_v, *bufsem):
        bufs = bufsem[:_NBUF]
        sems = bufsem[_NBUF:]
        wid = lax.axis_index("s") * _NC + lax.axis_index("c")
        base = wid * _BPW
        pltpu.sync_copy(idx_hbm.at[pl.ds(base, _BPW)], idx_v)
        copies = [
            pltpu.async_copy(
                table_hbm.at[idx_v.at[pl.ds(c * _CH, _CH)]],
                bufs[c % _NBUF],
                sems[c % _NBUF],
            )
            for c in range(_NBUF)
        ]
        for c in range(_NCHUNK):
            copies[c].wait()
            pltpu.sync_copy(bufs[c % _NBUF], out_hbm.at[pl.ds(base + c * _CH, _CH)])
            nxt = c + _NBUF
            if nxt < _NCHUNK:
                copies.append(
                    pltpu.async_copy(
                        table_hbm.at[idx_v.at[pl.ds(nxt * _CH, _CH)]],
                        bufs[nxt % _NBUF],
                        sems[nxt % _NBUF],
                    )
                )

    return _gather


_UT = 8

def _fused_body(len_ref, xp_ref, whh_ref, wout_ref, b_ref, out_ref, h_ref):
    i = pl.program_id(0)

    @pl.when(i == 0)
    def _():
        h_ref[...] = jnp.zeros_like(h_ref)

    h = h_ref[...]
    recs = []
    for j in range(_UT):
        t = i * _UT + j
        w = lax.bitcast_convert_type(xp_ref[pl.ds(j * B, B), :], jnp.uint32)
        x = jnp.concatenate(
            [
                lax.bitcast_convert_type(w << 16, jnp.float32),
                lax.bitcast_convert_type(w & jnp.uint32(0xFFFF0000), jnp.float32),
            ],
            axis=1,
        )
        h_new = jnp.maximum(
            x
            + jnp.dot(
                h.astype(jnp.bfloat16), whh_ref[...],
                preferred_element_type=jnp.float32,
            ),
            0.0,
        )
        mask = len_ref[...] > t
        h = jnp.where(mask, h_new, h)
        recs.append(jnp.where(mask, h_new, 0.0).astype(jnp.bfloat16))
    h_ref[...] = h

    rec = jnp.concatenate(recs, axis=0)
    xt = rec.T
    logits = (
        jnp.dot(wout_ref[...], xt, preferred_element_type=jnp.float32)
        + b_ref[...]
    )
    m = jnp.max(logits, axis=0, keepdims=True)
    e = jnp.exp(logits - m)
    s = jnp.sum(e, axis=0, keepdims=True)
    res = logits - m - jnp.log(s)
    for j in range(_UT):
        out_ref[j] = res[:, j * B:(j + 1) * B]


def _run_fused(lengths2d, xp, w_hh_t_bf16, w_out_bf16, b_out2d):
    return pl.pallas_call(
        _fused_body,
        grid=(T // _UT,),
        in_specs=[
            pl.BlockSpec((B, 1), lambda i: (0, 0)),
            pl.BlockSpec((_UT * B, _HH), lambda i: (i, 0)),
            pl.BlockSpec((HID, HID), lambda i: (0, 0)),
            pl.BlockSpec((IN_DIM, HID), lambda i: (0, 0)),
            pl.BlockSpec((IN_DIM, 1), lambda i: (0, 0)),
        ],
        out_specs=pl.BlockSpec((_UT, IN_DIM, B), lambda i: (i, 0, 0)),
        out_shape=jax.ShapeDtypeStruct((T, IN_DIM, B), jnp.float32),
        scratch_shapes=[pltpu.VMEM((B, HID), jnp.float32)],
    )(lengths2d, xp, w_hh_t_bf16, w_out_bf16, b_out2d)


def kernel(inputs, lengths, embed, W_ih, W_hh, b_ih, b_hh, W_out, b_out):
    emb_pad = jnp.pad(embed, ((0, VPAD - (IN_DIM + 1)), (0, 0)))
    bias2d = (b_ih + b_hh).reshape(1, HID)
    table = _make_table(emb_pad, W_ih.T, bias2d)

    idx = inputs.reshape(N).astype(jnp.int32)
    xp = _build_gather()(table, idx)

    lengths2d = lengths.reshape(B, 1).astype(jnp.int32)
    out_tvb = _run_fused(
        lengths2d, xp, W_hh.T.astype(jnp.bfloat16),
        W_out.astype(jnp.bfloat16), b_out.reshape(IN_DIM, 1),
    )
    return out_tvb.transpose(0, 2, 1)


# speedup vs baseline: 2.7284x; 1.0078x over previous
import functools

import jax
import jax.numpy as jnp
from jax import lax
from jax.experimental import pallas as pl
from jax.experimental.pallas import tpu as pltpu
from jax.experimental.pallas import tpu_sc as plsc

IN_DIM = 1000
EMBED = 512
HID = 512
T = 200
B = 128
N = T * B
VPAD = 1008
_HH = HID // 2

_TH = T // 2
_NH = _TH * B


def _table_body(emb_ref, w_ref, bias_ref, out_ref):
    res = (
        lax.dot_general(
            emb_ref[...], w_ref[...], (((1,), (1,)), ((), ())),
            preferred_element_type=jnp.float32,
        )
        + bias_ref[...]
    )
    lo = lax.bitcast_convert_type(res[:, :_HH], jnp.uint32)
    hi = lax.bitcast_convert_type(res[:, _HH:], jnp.uint32)
    lo_t = (lo + 0x8000) >> 16
    hi_t = (hi + 0x8000) & jnp.uint32(0xFFFF0000)
    out_ref[...] = lax.bitcast_convert_type(hi_t | lo_t, jnp.float32)


def _make_table(emb_pad, w_ih, bias2d):
    return pl.pallas_call(
        _table_body,
        out_shape=jax.ShapeDtypeStruct((VPAD, _HH), jnp.float32),
    )(emb_pad, w_ih, bias2d)


_NC = 2
_NS = 16
_NW = _NC * _NS
_BPW = _NH // _NW
_CH = 80
_NBUF = 4
_NCHUNK = _BPW // _CH


@functools.cache
def _build_gather():
    mesh = plsc.VectorSubcoreMesh(core_axis_name="c", subcore_axis_name="s")

    @functools.partial(
        pl.kernel,
        out_type=jax.ShapeDtypeStruct((_NH, _HH), jnp.float32),
        mesh=mesh,
        scratch_types=[
            pltpu.VMEM((_BPW,), jnp.int32),
        ]
        + [pltpu.VMEM((_CH, _HH), jnp.float32) for _ in range(_NBUF)]
        + [pltpu.SemaphoreType.DMA for _ in range(_NBUF)],
    )
    def _gather(table_hbm, idx_hbm, out_hbm, idx_v, *bufsem):
        bufs = bufsem[:_NBUF]
        sems = bufsem[_NBUF:]
        wid = lax.axis_index("s") * _NC + lax.axis_index("c")
        base = wid * _BPW
        pltpu.sync_copy(idx_hbm.at[pl.ds(base, _BPW)], idx_v)
        copies = [
            pltpu.async_copy(
                table_hbm.at[idx_v.at[pl.ds(c * _CH, _CH)]],
                bufs[c % _NBUF],
                sems[c % _NBUF],
            )
            for c in range(min(_NBUF, _NCHUNK))
        ]
        for c in range(_NCHUNK):
            copies[c].wait()
            pltpu.sync_copy(bufs[c % _NBUF], out_hbm.at[pl.ds(base + c * _CH, _CH)])
            nxt = c + _NBUF
            if nxt < _NCHUNK:
                copies.append(
                    pltpu.async_copy(
                        table_hbm.at[idx_v.at[pl.ds(nxt * _CH, _CH)]],
                        bufs[nxt % _NBUF],
                        sems[nxt % _NBUF],
                    )
                )

    return _gather


_UT = 10
_GRID_H = _TH // _UT


def _fused_body(t_off, len_ref, xp_ref, whh_ref, wout_ref, b_ref, *rest):
    if t_off == 0:
        out_ref, hout_ref, h_ref = rest
    else:
        hin_ref, _outal_ref, out_ref, h_ref = rest
    i = pl.program_id(0)

    @pl.when(i == 0)
    def _():
        if t_off == 0:
            h_ref[...] = jnp.zeros_like(h_ref)
        else:
            h_ref[...] = hin_ref[...]

    h = h_ref[...]
    recs = []
    for j in range(_UT):
        t = t_off + i * _UT + j
        w = lax.bitcast_convert_type(xp_ref[pl.ds(j * B, B), :], jnp.uint32)
        x = jnp.concatenate(
            [
                lax.bitcast_convert_type(w << 16, jnp.float32),
                lax.bitcast_convert_type(w & jnp.uint32(0xFFFF0000), jnp.float32),
            ],
            axis=1,
        )
        h_new = jnp.maximum(
            x
            + lax.dot_general(
                h.astype(jnp.bfloat16), whh_ref[...], (((1,), (1,)), ((), ())),
                preferred_element_type=jnp.float32,
            ),
            0.0,
        )
        mask = len_ref[...] > t
        h = jnp.where(mask, h_new, h)
        recs.append(jnp.where(mask, h_new, 0.0).astype(jnp.bfloat16))
    h_ref[...] = h
    if t_off == 0:
        hout_ref[...] = h

    rec = jnp.concatenate(recs, axis=0)
    xt = rec.T
    logits = (
        jnp.dot(wout_ref[...], xt, preferred_element_type=jnp.float32)
        + b_ref[...]
    )
    m = jnp.max(logits, axis=0, keepdims=True)
    e = jnp.exp(logits - m)
    s = jnp.sum(e, axis=0, keepdims=True)
    res = logits - m - jnp.log(s)
    for j in range(_UT):
        out_ref[j] = res[:, j * B:(j + 1) * B]


_W_SPECS = [
    pl.BlockSpec((B, 1), lambda i: (0, 0)),
    pl.BlockSpec((_UT * B, _HH), lambda i: (i, 0)),
    pl.BlockSpec((HID, HID), lambda i: (0, 0)),
    pl.BlockSpec((IN_DIM, HID), lambda i: (0, 0)),
    pl.BlockSpec((IN_DIM, 1), lambda i: (0, 0)),
]


def _run_fused_a(lengths2d, xp1, w_hh_bf16, w_out_bf16, b_out2d):
    return pl.pallas_call(
        functools.partial(_fused_body, 0),
        grid=(_GRID_H,),
        in_specs=_W_SPECS,
        out_specs=[
            pl.BlockSpec((_UT, IN_DIM, B), lambda i: (i, 0, 0)),
            pl.BlockSpec((B, HID), lambda i: (0, 0)),
        ],
        out_shape=[
            jax.ShapeDtypeStruct((T, IN_DIM, B), jnp.float32),
            jax.ShapeDtypeStruct((B, HID), jnp.float32),
        ],
        scratch_shapes=[pltpu.VMEM((B, HID), jnp.float32)],
    )(lengths2d, xp1, w_hh_bf16, w_out_bf16, b_out2d)


def _run_fused_b(lengths2d, xp2, w_hh_bf16, w_out_bf16, b_out2d, h_mid, out_alias):
    return pl.pallas_call(
        functools.partial(_fused_body, _TH),
        grid=(_GRID_H,),
        in_specs=_W_SPECS
        + [
            pl.BlockSpec((B, HID), lambda i: (0, 0)),
            pl.BlockSpec(memory_space=pl.ANY),
        ],
        out_specs=pl.BlockSpec((_UT, IN_DIM, B), lambda i: (i + _GRID_H, 0, 0)),
        out_shape=jax.ShapeDtypeStruct((T, IN_DIM, B), jnp.float32),
        input_output_aliases={6: 0},
        scratch_shapes=[pltpu.VMEM((B, HID), jnp.float32)],
    )(lengths2d, xp2, w_hh_bf16, w_out_bf16, b_out2d, h_mid, out_alias)


def kernel(inputs, lengths, embed, W_ih, W_hh, b_ih, b_hh, W_out, b_out):
    emb_pad = jnp.pad(embed, ((0, VPAD - (IN_DIM + 1)), (0, 0)))
    bias2d = (b_ih + b_hh).reshape(1, HID)
    table = _make_table(emb_pad, W_ih, bias2d)

    idx = inputs.reshape(N).astype(jnp.int32)
    gather = _build_gather()
    xp1 = gather(table, idx[:_NH])
    xp2 = gather(table, idx[_NH:])

    lengths2d = lengths.reshape(B, 1).astype(jnp.int32)
    w_hh_bf16 = W_hh.astype(jnp.bfloat16)
    w_out_bf16 = W_out.astype(jnp.bfloat16)
    b_out2d = b_out.reshape(IN_DIM, 1)

    out_a, h_mid = _run_fused_a(lengths2d, xp1, w_hh_bf16, w_out_bf16, b_out2d)
    out_tvb = _run_fused_b(
        lengths2d, xp2, w_hh_bf16, w_out_bf16, b_out2d, h_mid, out_a
    )
    return out_tvb.transpose(0, 2, 1)
